# initial kernel scaffold (unmeasured)
import functools

import numpy as np
import jax
import jax.numpy as jnp
from jax import lax
from jax.experimental import pallas as pl
from jax.experimental.pallas import tpu as pltpu

N_DEV = 32
B_LOC = 2
SQ = 128
D = 512
H_LOC = 4
DH = 64
ROWS = B_LOC * SQ
HD_LOC = H_LOC * DH

_sem_signal = getattr(pltpu, "semaphore_signal", None) or getattr(pl, "semaphore_signal")
_sem_wait = getattr(pltpu, "semaphore_wait", None) or getattr(pl, "semaphore_wait")
_CompilerParams = getattr(pltpu, "CompilerParams", None) or getattr(
    pltpu, "TPUCompilerParams"
)


def _rope_tables():
    inv = 1.0 / (10000.0 ** (np.arange(0, DH, 2) / DH))
    pos = np.arange(SQ)[:, None] * inv[None, :]
    cos = np.repeat(np.cos(pos), 2, axis=-1).astype(np.float32)
    sin = np.repeat(np.sin(pos), 2, axis=-1).astype(np.float32)
    R = np.zeros((DH, DH), np.float32)
    k = np.arange(0, DH, 2)
    R[k + 1, k] = -1.0
    R[k, k + 1] = 1.0
    return cos, sin, R


def kernel(x, Wq, Wk, Wv, Wo):
    cos_np, sin_np, R_np = _rope_tables()
    cos_in = jnp.asarray(cos_np)
    sin_in = jnp.asarray(sin_np)
    R_in = jnp.asarray(R_np)

    def body(x_ref, wq_ref, wk_ref, wv_ref, wo_ref, cos_ref, sin_ref, r_ref,
             out_ref, comm_ref, send_sems, recv_sems, credit_sem):
        my = lax.axis_index("i")
        left = lax.rem(my - 1 + N_DEV, N_DEV)
        right = lax.rem(my + 1, N_DEV)

        barrier_sem = pltpu.get_barrier_semaphore()
        for nbr in (left, right):
            _sem_signal(barrier_sem, inc=1, device_id=(nbr,),
                        device_id_type=pl.DeviceIdType.MESH)
        _sem_wait(barrier_sem, 2)

        wq = wq_ref[...]
        wk = wk_ref[...]
        wv = wv_ref[...]
        wo = wo_ref[...]
        cos = cos_ref[...]
        sin = sin_ref[...]
        R = r_ref[...]

        def partial_for(xc):
            q = jnp.dot(xc, wq, preferred_element_type=jnp.float32)
            k = jnp.dot(xc, wk, preferred_element_type=jnp.float32)
            v = jnp.dot(xc, wv, preferred_element_type=jnp.float32)
            parts = []
            for b in range(B_LOC):
                pb = jnp.zeros((SQ, D), jnp.float32)
                for h in range(H_LOC):
                    rs = slice(b * SQ, (b + 1) * SQ)
                    cs = slice(h * DH, (h + 1) * DH)
                    qb = q[rs, cs]
                    kb = k[rs, cs]
                    vb = v[rs, cs]
                    qr = qb * cos + jnp.dot(qb, R, preferred_element_type=jnp.float32) * sin
                    kr = kb * cos + jnp.dot(kb, R, preferred_element_type=jnp.float32) * sin
                    sc = lax.dot_general(
                        qr, kr, (((1,), (1,)), ((), ())),
                        preferred_element_type=jnp.float32,
                    ) * 0.125
                    m = jnp.max(sc, axis=1, keepdims=True)
                    e = jnp.exp(sc - m)
                    w = e / jnp.sum(e, axis=1, keepdims=True)
                    ctx = jnp.dot(w, vb, preferred_element_type=jnp.float32)
                    pb = pb + jnp.dot(ctx, wo[h * DH:(h + 1) * DH, :],
                                      preferred_element_type=jnp.float32)
                parts.append(pb)
            return jnp.concatenate(parts, axis=0)

        def hop_send(s, r):
            rdma = pltpu.make_async_remote_copy(
                src_ref=comm_ref.at[s],
                dst_ref=comm_ref.at[r],
                send_sem=send_sems.at[s],
                recv_sem=recv_sems.at[r],
                device_id=(right,),
                device_id_type=pl.DeviceIdType.MESH,
            )
            rdma.start()
            rdma.wait()

        xc0 = x_ref[...].reshape(ROWS, D)
        comm_ref[0, 0] = xc0
        comm_ref[0, 1] = partial_for(xc0)
        hop_send(0, 1)
        _sem_signal(credit_sem, inc=1, device_id=(left,),
                    device_id_type=pl.DeviceIdType.MESH)

        def loop_body(h, carry):
            s = lax.rem(h, 2)
            r = 1 - s
            xc = comm_ref[s, 0]
            comm_ref[s, 1] = comm_ref[s, 1] + partial_for(xc)
            _sem_wait(credit_sem, 1)
            hop_send(s, r)

            @pl.when(h < N_DEV - 1)
            def _():
                _sem_signal(credit_sem, inc=1, device_id=(left,),
                            device_id_type=pl.DeviceIdType.MESH)

            return carry

        lax.fori_loop(1, N_DEV, loop_body, 0)

        out_ref[...] = comm_ref[0, 1].reshape(B_LOC, SQ, D)

    out_shape = jax.ShapeDtypeStruct((B_LOC, SQ, D), jnp.float32)
    vmem = pl.BlockSpec(memory_space=pltpu.ANY) if False else pl.BlockSpec(
        memory_space=pltpu.VMEM
    )
    return pl.pallas_call(
        body,
        out_shape=out_shape,
        in_specs=[vmem] * 8,
        out_specs=vmem,
        scratch_shapes=[
            pltpu.VMEM((2, 2, ROWS, D), jnp.float32),
            pltpu.SemaphoreType.DMA((2,)),
            pltpu.SemaphoreType.DMA((2,)),
            pltpu.SemaphoreType.REGULAR,
        ],
        compiler_params=_CompilerParams(collective_id=0),
    )(x, Wq, Wk, Wv, Wo, cos_in, sin_in, R_in)


# baseline (device time: 312940 ns/iter reference)
import numpy as np
import jax
import jax.numpy as jnp
from jax import lax
from jax.experimental import pallas as pl
from jax.experimental.pallas import tpu as pltpu

N_DEV = 32
B_LOC = 2
SQ = 128
D = 512
H_LOC = 4
DH = 64
ROWS = B_LOC * SQ

WIRE_DTYPE = jnp.bfloat16

_sem_signal = getattr(pltpu, "semaphore_signal", None) or getattr(pl, "semaphore_signal")
_sem_wait = getattr(pltpu, "semaphore_wait", None) or getattr(pl, "semaphore_wait")
_CompilerParams = getattr(pltpu, "CompilerParams", None) or getattr(
    pltpu, "TPUCompilerParams"
)


def _rope_tables():
    inv = 1.0 / (10000.0 ** (np.arange(0, DH, 2) / DH))
    pos = np.arange(SQ)[:, None] * inv[None, :]
    cos = np.repeat(np.cos(pos), 2, axis=-1).astype(np.float32)
    sin = np.repeat(np.sin(pos), 2, axis=-1).astype(np.float32)
    R = np.zeros((DH, DH), np.float32)
    k = np.arange(0, DH, 2)
    R[k + 1, k] = -1.0
    R[k, k + 1] = 1.0
    return cos, sin, R


def kernel(x, Wq, Wk, Wv, Wo):
    cos_np, sin_np, R_np = _rope_tables()
    cos_in = jnp.asarray(cos_np)
    sin_in = jnp.asarray(sin_np)
    R_in = jnp.asarray(R_np)

    def body(x_ref, wq_ref, wk_ref, wv_ref, wo_ref, cos_ref, sin_ref, r_ref,
             out_ref, xbuf, abuf, x_ssem, x_rsem, a_ssem, a_rsem,
             x_credit, a_credit):
        my = lax.axis_index("i")
        left = lax.rem(my - 1 + N_DEV, N_DEV)
        right = lax.rem(my + 1, N_DEV)

        barrier_sem = pltpu.get_barrier_semaphore()
        for nbr in (left, right):
            _sem_signal(barrier_sem, inc=1, device_id=(nbr,),
                        device_id_type=pl.DeviceIdType.MESH)
        _sem_wait(barrier_sem, 2)

        wq = wq_ref[...]
        wk = wk_ref[...]
        wv = wv_ref[...]
        wo = wo_ref[...]
        cos = cos_ref[...]
        sin = sin_ref[...]
        R = r_ref[...]

        def partial_for(xc):
            q = jnp.dot(xc, wq, preferred_element_type=jnp.float32)
            k = jnp.dot(xc, wk, preferred_element_type=jnp.float32)
            v = jnp.dot(xc, wv, preferred_element_type=jnp.float32)
            parts = []
            for b in range(B_LOC):
                pb = jnp.zeros((SQ, D), jnp.float32)
                for h in range(H_LOC):
                    rs = slice(b * SQ, (b + 1) * SQ)
                    cs = slice(h * DH, (h + 1) * DH)
                    qb = q[rs, cs]
                    kb = k[rs, cs]
                    vb = v[rs, cs]
                    qr = qb * cos + jnp.dot(qb, R, preferred_element_type=jnp.float32) * sin
                    kr = kb * cos + jnp.dot(kb, R, preferred_element_type=jnp.float32) * sin
                    sc = lax.dot_general(
                        qr, kr, (((1,), (1,)), ((), ())),
                        preferred_element_type=jnp.float32,
                    ) * 0.125
                    m = jnp.max(sc, axis=1, keepdims=True)
                    e = jnp.exp(sc - m)
                    w = e / jnp.sum(e, axis=1, keepdims=True)
                    ctx = jnp.dot(w, vb, preferred_element_type=jnp.float32)
                    pb = pb + jnp.dot(ctx, wo[h * DH:(h + 1) * DH, :],
                                      preferred_element_type=jnp.float32)
                parts.append(pb)
            return jnp.concatenate(parts, axis=0)

        def mk(buf, s_, r_, ssem, rsem):
            return pltpu.make_async_remote_copy(
                src_ref=buf.at[s_],
                dst_ref=buf.at[r_],
                send_sem=ssem.at[s_],
                recv_sem=rsem.at[r_],
                device_id=(right,),
                device_id_type=pl.DeviceIdType.MESH,
            )

        def signal(sem, nbr):
            _sem_signal(sem, inc=1, device_id=(nbr,),
                        device_id_type=pl.DeviceIdType.MESH)

        xc0 = x_ref[...].reshape(ROWS, D)
        xbuf[0] = xc0.astype(WIRE_DTYPE)
        x_send0 = mk(xbuf, 0, 1, x_ssem, x_rsem)
        x_send0.start()
        abuf[0] = partial_for(xc0).astype(WIRE_DTYPE)
        a_send0 = mk(abuf, 0, 1, a_ssem, a_rsem)
        a_send0.start()
        x_send0.wait_send()
        signal(x_credit, left)
        a_send0.wait_send()
        signal(a_credit, left)

        def step(t, carry):
            s = lax.rem(t, 2)
            r = 1 - s
            x_recv = mk(xbuf, r, s, x_ssem, x_rsem)
            x_recv.wait_recv()
            x_send = mk(xbuf, s, r, x_ssem, x_rsem)

            @pl.when(t < N_DEV - 1)
            def _():
                _sem_wait(x_credit, 1)
                x_send.start()

            p = partial_for(xbuf[s].astype(jnp.float32))

            a_recv = mk(abuf, r, s, a_ssem, a_rsem)
            a_recv.wait_recv()
            abuf[s] = (abuf[s].astype(jnp.float32) + p).astype(WIRE_DTYPE)
            _sem_wait(a_credit, 1)
            a_send = mk(abuf, s, r, a_ssem, a_rsem)
            a_send.start()

            @pl.when(t < N_DEV - 1)
            def _():
                x_send.wait_send()

            @pl.when(t < N_DEV - 2)
            def _():
                signal(x_credit, left)

            a_send.wait_send()

            @pl.when(t < N_DEV - 1)
            def _():
                signal(a_credit, left)

            return carry

        lax.fori_loop(1, N_DEV, step, 0)

        a_final = mk(abuf, 1, 0, a_ssem, a_rsem)
        a_final.wait_recv()
        out_ref[...] = abuf[0].astype(jnp.float32).reshape(B_LOC, SQ, D)

    out_shape = jax.ShapeDtypeStruct((B_LOC, SQ, D), jnp.float32)
    vmem = pl.BlockSpec(memory_space=pltpu.VMEM)
    return pl.pallas_call(
        body,
        out_shape=out_shape,
        in_specs=[vmem] * 8,
        out_specs=vmem,
        scratch_shapes=[
            pltpu.VMEM((2, ROWS, D), WIRE_DTYPE),
            pltpu.VMEM((2, ROWS, D), WIRE_DTYPE),
            pltpu.SemaphoreType.DMA((2,)),
            pltpu.SemaphoreType.DMA((2,)),
            pltpu.SemaphoreType.DMA((2,)),
            pltpu.SemaphoreType.DMA((2,)),
            pltpu.SemaphoreType.REGULAR,
            pltpu.SemaphoreType.REGULAR,
        ],
        compiler_params=_CompilerParams(collective_id=0),
    )(x, Wq, Wk, Wv, Wo, cos_in, sin_in, R_in)
